# baseline (speedup 1.0000x reference)
import functools
import jax
import jax.numpy as jnp
from jax import lax
from jax.experimental import pallas as pl
from jax.experimental.pallas import tpu as pltpu
from jax.experimental.pallas import tpu_sc as plsc

BATCH = 16384
SEQ = 5
DIM = 64
NROWS = 1000000
LANES = 16
NC, NS = 2, 16
NW = NC * NS
BPW = BATCH // NW
G = 128
KG = BPW // G
BLK = 65536
PARAMS_LEN = 16


def _proj_body(wm_ref, tt_ref, *out_refs):
    res = jax.lax.dot_general(
        wm_ref[...], tt_ref[...], (((1,), (0,)), ((), ())),
        preferred_element_type=jnp.float32)
    for l, o in enumerate(out_refs):
        o[...] = res[l]


@jax.jit
def _proj(wm, tt):
    grid = (NROWS + BLK - 1) // BLK
    return pl.pallas_call(
        _proj_body,
        grid=(grid,),
        in_specs=[
            pl.BlockSpec((8, DIM), lambda i: (0, 0)),
            pl.BlockSpec((DIM, BLK), lambda i: (0, i)),
        ],
        out_specs=[pl.BlockSpec((BLK,), lambda i: (i,)) for _ in range(SEQ)],
        out_shape=[jax.ShapeDtypeStruct((NROWS,), jnp.float32)
                   for _ in range(SEQ)],
    )(wm, tt)


def _sc_body(x3_hbm, p0_hbm, p1_hbm, p2_hbm, p3_hbm, p4_hbm, params_hbm,
             out_hbm, idx_v, g_v, db_v, out_v, sem):
    wid = lax.axis_index("s") * NC + lax.axis_index("c")
    proj = (p0_hbm, p1_hbm, p2_hbm, p3_hbm, p4_hbm)

    pltpu.sync_copy(params_hbm, db_v)
    for l in range(SEQ):
        pltpu.sync_copy(x3_hbm.at[l, pl.ds(wid * KG, KG)], idx_v.at[l])

    copies = []
    for l in range(SEQ):
        for k in range(KG):
            copies.append(pltpu.async_copy(
                proj[l].at[idx_v.at[l, k]], g_v.at[l, k], sem))
    for c in copies:
        c.wait()

    db = db_v[...]
    ones = jnp.zeros((LANES,), jnp.float32) + 1.0
    for k in range(KG):
        for ig in range(G // LANES):
            sl = pl.ds(ig * LANES, LANES)
            delta = g_v[0, k, sl] + g_v[1, k, sl] + g_v[2, k, sl] \
                + g_v[3, k, sl] + g_v[4, k, sl] + db
            p0 = ones / (ones + jnp.exp(-delta))
            off = k * G + ig * LANES
            out_v[0, pl.ds(off, LANES)] = p0
            out_v[1, pl.ds(off, LANES)] = ones - p0

    base = wid * BPW
    pltpu.sync_copy(out_v.at[0], out_hbm.at[0, pl.ds(base, BPW)])
    pltpu.sync_copy(out_v.at[1], out_hbm.at[1, pl.ds(base, BPW)])


@jax.jit
def _run(x3, p0, p1, p2, p3, p4, params):
    mesh = plsc.VectorSubcoreMesh(core_axis_name="c", subcore_axis_name="s")
    f = pl.kernel(
        _sc_body,
        out_type=jax.ShapeDtypeStruct((2, BATCH), jnp.float32),
        mesh=mesh,
        scratch_types=[
            pltpu.VMEM((SEQ, KG, G), jnp.int32),
            pltpu.VMEM((SEQ, KG, G), jnp.float32),
            pltpu.VMEM((PARAMS_LEN,), jnp.float32),
            pltpu.VMEM((2, BPW), jnp.float32),
            pltpu.SemaphoreType.DMA,
        ],
        compiler_params=pltpu.CompilerParams(
            needs_layout_passes=False, use_tc_tiling_on_sc=False),
    )
    return f(x3, p0, p1, p2, p3, p4, params)


def kernel(x, table, W, b):
    dw = W[0] - W[1]
    wm = jnp.zeros((8, DIM), jnp.float32).at[:SEQ].set(dw.reshape(SEQ, DIM))
    projs = _proj(wm, table.T)
    x3 = x.astype(jnp.int32).T.reshape(SEQ, BATCH // G, G)
    params = jnp.full((PARAMS_LEN,), b[0] - b[1], jnp.float32)
    out2 = _run(x3, *projs, params)
    return out2.T


# speedup vs baseline: 1.0238x; 1.0238x over previous
import functools
import jax
import jax.numpy as jnp
from jax import lax
from jax.experimental import pallas as pl
from jax.experimental.pallas import tpu as pltpu
from jax.experimental.pallas import tpu_sc as plsc

BATCH = 16384
SEQ = 5
DIM = 64
NROWS = 1000000
LANES = 16
NC, NS = 2, 16
NW = NC * NS
BPW = BATCH // NW
G = 128
KG = BPW // G
BLK = 32768
PARAMS_LEN = 16


def _proj_body(wm_ref, tt_ref, *out_refs):
    res = jax.lax.dot_general(
        wm_ref[...], tt_ref[...], (((1,), (0,)), ((), ())),
        preferred_element_type=jnp.float32)
    for l, o in enumerate(out_refs):
        o[...] = res[l]


@jax.jit
def _proj(wm, tt):
    grid = (NROWS + BLK - 1) // BLK
    return pl.pallas_call(
        _proj_body,
        grid=(grid,),
        in_specs=[
            pl.BlockSpec((8, DIM), lambda i: (0, 0)),
            pl.BlockSpec((DIM, BLK), lambda i: (0, i)),
        ],
        out_specs=[pl.BlockSpec((BLK,), lambda i: (i,)) for _ in range(SEQ)],
        out_shape=[jax.ShapeDtypeStruct((NROWS,), jnp.float32)
                   for _ in range(SEQ)],
    )(wm, tt)


def _sc_body(x3_hbm, p0_hbm, p1_hbm, p2_hbm, p3_hbm, p4_hbm, params_hbm,
             out_hbm, idx_v, g_v, db_v, out_v, sem):
    wid = lax.axis_index("s") * NC + lax.axis_index("c")
    proj = (p0_hbm, p1_hbm, p2_hbm, p3_hbm, p4_hbm)

    pltpu.sync_copy(params_hbm, db_v)
    for l in range(SEQ):
        pltpu.sync_copy(x3_hbm.at[l, pl.ds(wid * KG, KG)], idx_v.at[l])

    copies = []
    for l in range(SEQ):
        for k in range(KG):
            copies.append(pltpu.async_copy(
                proj[l].at[idx_v.at[l, k]], g_v.at[l, k], sem))
    for c in copies:
        c.wait()

    db = db_v[...]
    ones = jnp.zeros((LANES,), jnp.float32) + 1.0
    for k in range(KG):
        for ig in range(G // LANES):
            sl = pl.ds(ig * LANES, LANES)
            delta = g_v[0, k, sl] + g_v[1, k, sl] + g_v[2, k, sl] \
                + g_v[3, k, sl] + g_v[4, k, sl] + db
            p0 = ones / (ones + jnp.exp(-delta))
            off = k * G + ig * LANES
            out_v[0, pl.ds(off, LANES)] = p0
            out_v[1, pl.ds(off, LANES)] = ones - p0

    base = wid * BPW
    pltpu.sync_copy(out_v.at[0], out_hbm.at[0, pl.ds(base, BPW)])
    pltpu.sync_copy(out_v.at[1], out_hbm.at[1, pl.ds(base, BPW)])


@jax.jit
def _run(x3, p0, p1, p2, p3, p4, params):
    mesh = plsc.VectorSubcoreMesh(core_axis_name="c", subcore_axis_name="s")
    f = pl.kernel(
        _sc_body,
        out_type=jax.ShapeDtypeStruct((2, BATCH), jnp.float32),
        mesh=mesh,
        scratch_types=[
            pltpu.VMEM((SEQ, KG, G), jnp.int32),
            pltpu.VMEM((SEQ, KG, G), jnp.float32),
            pltpu.VMEM((PARAMS_LEN,), jnp.float32),
            pltpu.VMEM((2, BPW), jnp.float32),
            pltpu.SemaphoreType.DMA,
        ],
        compiler_params=pltpu.CompilerParams(
            needs_layout_passes=False, use_tc_tiling_on_sc=False),
    )
    return f(x3, p0, p1, p2, p3, p4, params)


def kernel(x, table, W, b):
    dw = W[0] - W[1]
    wm = jnp.zeros((8, DIM), jnp.float32).at[:SEQ].set(dw.reshape(SEQ, DIM))
    projs = _proj(wm, table.T)
    x3 = x.astype(jnp.int32).T.reshape(SEQ, BATCH // G, G)
    params = jnp.full((PARAMS_LEN,), b[0] - b[1], jnp.float32)
    out2 = _run(x3, *projs, params)
    return out2.T
